# baseline (speedup 1.0000x reference)
import jax
import jax.numpy as jnp
from jax import lax
from jax.experimental import pallas as pl
from jax.experimental.pallas import tpu as pltpu
from jax.experimental.pallas import tpu_sc as plsc

B, S, T, D = 16, 4096, 4096, 1024
L = 16
NC, NS = 2, 16
NW = NC * NS
ROWS_PER_W = (B * T) // NW
K = 32
NCHUNK = ROWS_PER_W // K
VPC = K // L
MAXQ = 8


def _sc_body(enc_hbm, steps_hbm, tgt_hbm, out_hbm, mask_hbm,
             steps_v, tgt_v, idx_a, idx_b, rows_a, rows_b, mask_v,
             gsem_a, gsem_b, wsem):
    wid = lax.axis_index("s") * NC + lax.axis_index("c")
    b = wid // 2
    h = (wid + b) % 2
    t0 = h * ROWS_PER_W
    row0 = b * T + t0

    pltpu.sync_copy(steps_hbm.at[wid], steps_v)
    pltpu.sync_copy(tgt_hbm.at[wid], tgt_v)
    steps = steps_v[...]
    tgt = tgt_v[...]
    tgt_s = tgt[0]
    base_flat = jnp.full((L,), b * S, jnp.int32)

    def mask_chunk(c, carry):
        for j in range(VPC):
            t_i = t0 + c * K + j * L + lax.iota(jnp.int32, 16)
            mask_v[pl.ds(c * K + j * L, L)] = jnp.where(
                t_i < tgt, jnp.float32(1.0), jnp.float32(0.0))
        return carry

    lax.fori_loop(0, NCHUNK, mask_chunk, 0)
    pltpu.sync_copy(mask_v, mask_hbm.at[pl.ds(row0, ROWS_PER_W)])

    def compute_chunk(c, idx_ref):
        for j in range(VPC):
            t_i = t0 + c * K + j * L + lax.iota(jnp.int32, 16)
            x = steps * t_i.astype(jnp.float32)
            f = x.astype(jnp.int32)
            fr = x - f.astype(jnp.float32)
            half = jnp.float32(0.5)
            odd = (f & 1) == 1
            inc = jnp.where((fr > half) | ((fr == half) & odd), 1, 0)
            r = jnp.minimum(jnp.maximum(f + inc, 0), S - 1)
            m = t_i < tgt
            idx_ref[pl.ds(j * L, L)] = base_flat + jnp.where(m, r, 1)

    def gather(idx_ref, rows_ref, sem):
        return pltpu.make_async_copy(enc_hbm.at[idx_ref], rows_ref, sem)

    def wb(c, rows_ref):
        pltpu.sync_copy(rows_ref, out_hbm.at[pl.ds(row0 + c * K, K)])

    n_un = jnp.minimum(jnp.maximum(tgt_s - t0, 0), ROWS_PER_W)
    nch = (n_un + (K - 1)) // K
    npairs = (nch + 1) // 2
    c_tail = 2 * npairs
    n_tail = NCHUNK - c_tail

    @pl.when(npairs > 0)
    def _prologue():
        compute_chunk(0, idx_a)
        gather(idx_a, rows_a, gsem_a).start()

    def pair(k, carry):
        c0 = 2 * k
        compute_chunk(c0 + 1, idx_b)
        gather(idx_b, rows_b, gsem_b).start()
        gather(idx_a, rows_a, gsem_a).wait()
        wb(c0, rows_a)
        compute_chunk(c0 + 2, idx_a)
        gather(idx_a, rows_a, gsem_a).start()
        gather(idx_b, rows_b, gsem_b).wait()
        wb(c0 + 1, rows_b)
        return carry

    lax.fori_loop(0, npairs - 1, pair, 0)

    @pl.when(npairs > 0)
    def _epilogue():
        c0 = 2 * npairs - 2
        compute_chunk(c0 + 1, idx_b)
        gather(idx_b, rows_b, gsem_b).start()
        gather(idx_a, rows_a, gsem_a).wait()
        wb(c0, rows_a)
        gather(idx_b, rows_b, gsem_b).wait()
        wb(c0 + 1, rows_b)

    @pl.when(n_tail > 0)
    def _fill():
        for j in range(VPC):
            idx_a[pl.ds(j * L, L)] = base_flat + 1
        gather(idx_a, rows_a, gsem_a).start()
        gather(idx_a, rows_a, gsem_a).wait()

    def wb_async():
        return pltpu.make_async_copy(
            rows_a, out_hbm.at[pl.ds(row0, K)], wsem)

    def tail_wb(c, carry):
        pltpu.make_async_copy(
            rows_a, out_hbm.at[pl.ds(row0 + c * K, K)], wsem).start()

        @pl.when(c >= c_tail + MAXQ)
        def _():
            wb_async().wait()
        return carry

    lax.fori_loop(c_tail, NCHUNK, tail_wb, 0)

    def drain(j, carry):
        wb_async().wait()
        return carry

    lax.fori_loop(0, jnp.minimum(n_tail, MAXQ), drain, 0)


@jax.jit
def _sc_call(enc2, steps_w, tgt_w):
    mesh = plsc.VectorSubcoreMesh(core_axis_name="c", subcore_axis_name="s")
    fn = pl.kernel(
        _sc_body,
        out_type=(
            jax.ShapeDtypeStruct((B * T, D), jnp.float32),
            jax.ShapeDtypeStruct((B * T,), jnp.float32),
        ),
        mesh=mesh,
        scratch_types=[
            pltpu.VMEM((L,), jnp.float32),
            pltpu.VMEM((L,), jnp.int32),
            pltpu.VMEM((K,), jnp.int32),
            pltpu.VMEM((K,), jnp.int32),
            pltpu.VMEM((K, D), jnp.float32),
            pltpu.VMEM((K, D), jnp.float32),
            pltpu.VMEM((ROWS_PER_W,), jnp.float32),
            pltpu.SemaphoreType.DMA,
            pltpu.SemaphoreType.DMA,
            pltpu.SemaphoreType.DMA,
        ],
    )
    return fn(enc2, steps_w, tgt_w)


def kernel(enc, src_lens, tgt_lens):
    enc2 = enc.reshape(B * S, D)
    steps = (src_lens.astype(jnp.float32) - 1.0) / (
        tgt_lens.astype(jnp.float32) - 1.0)
    steps_w = jnp.broadcast_to(jnp.repeat(steps, NW // B)[:, None], (NW, L))
    tgt_w = jnp.broadcast_to(
        jnp.repeat(tgt_lens.astype(jnp.int32), NW // B)[:, None], (NW, L))
    out, masks = _sc_call(enc2, steps_w, tgt_w)
    return out.reshape(B, T, D), masks.reshape(B, T)
